# baseline (speedup 1.0000x reference)
import functools
import numpy as np
import jax
import jax.numpy as jnp
from jax import lax
from jax.experimental import pallas as pl
from jax.experimental.pallas import tpu as pltpu
from jax.experimental.pallas import tpu_sc as plsc

_B, _L, _D = 16384, 200, 32
_N = _B * _L
_NW = 32
_PER_W = _N // _NW
_C = 2048
_CHUNKS = _PER_W // _C
_GN = 128
_LANES = 16

_BINS = np.linspace(-3.15, 3.15, 64).astype(np.float32)
_FMAX = np.finfo(np.float32).max
_EBHI = np.concatenate([_BINS, [_FMAX] * 64]).astype(np.float32)
_EBLO = np.concatenate([[-_FMAX], _BINS, [_FMAX] * 63]).astype(np.float32)
_LO = np.float32(_BINS[0])
_INV = np.float32(1.0 / ((3.15 - (-3.15)) / 63))


def _sc_body(x_hbm, table_hbm, eblo_hbm, ebhi_hbm, out_hbm,
             x_v, idx_v, rows_v, eblo_v, ebhi_v, sem):
    wid = lax.axis_index("s") * 2 + lax.axis_index("c")
    base = wid * _PER_W

    pltpu.sync_copy(eblo_hbm, eblo_v)
    pltpu.sync_copy(ebhi_hbm, ebhi_v)

    def chunk_body(ci, carry):
        off = base + ci * _C
        pltpu.sync_copy(x_hbm.at[pl.ds(off, _C)], x_v)

        def vec_body(i, c2):
            xv = x_v[pl.ds(i * _LANES, _LANES)]
            p = (xv - _LO) * _INV
            p = jnp.clip(p, -100.0, 100.0)
            te = jnp.clip(p.astype(jnp.int32) + 1, 0, 64)
            bhi = plsc.load_gather(ebhi_v, [te])
            blo = plsc.load_gather(eblo_v, [te])
            t = te + (xv >= bhi).astype(jnp.int32) - (xv < blo).astype(jnp.int32)
            idx = jnp.clip(t, 1, 64)
            idx = jnp.where(xv != xv, 0, idx)
            idx_v[pl.ds(i * _LANES, _LANES)] = idx
            return c2

        lax.fori_loop(0, _C // _LANES, vec_body, 0)

        pltpu.async_copy(table_hbm.at[idx_v], rows_v, sem).wait()

        pltpu.sync_copy(rows_v, out_hbm.at[pl.ds(off, _C)])
        return carry

    lax.fori_loop(0, _CHUNKS, chunk_body, 0)


@jax.jit
def kernel(x, table):
    mesh = plsc.VectorSubcoreMesh(core_axis_name="c", subcore_axis_name="s")
    call = pl.kernel(
        _sc_body,
        out_type=jax.ShapeDtypeStruct((_N, _D), jnp.float32),
        mesh=mesh,
        compiler_params=pltpu.CompilerParams(
            needs_layout_passes=False, use_tc_tiling_on_sc=False),
        scratch_types=[
            pltpu.VMEM((_C,), jnp.float32),
            pltpu.VMEM((_C,), jnp.int32),
            pltpu.VMEM((_C, _D), jnp.float32),
            pltpu.VMEM((128,), jnp.float32),
            pltpu.VMEM((128,), jnp.float32),
            pltpu.SemaphoreType.DMA,
        ],
    )
    out = call(x.reshape(_N), table, jnp.asarray(_EBLO), jnp.asarray(_EBHI))
    return out.reshape(_B, _L, _D)


# speedup vs baseline: 3.2864x; 3.2864x over previous
import functools
import numpy as np
import jax
import jax.numpy as jnp
from jax import lax
from jax.experimental import pallas as pl
from jax.experimental.pallas import tpu as pltpu
from jax.experimental.pallas import tpu_sc as plsc

_B, _L, _D = 16384, 200, 32
_N = _B * _L
_NW = 32
_PER_W = _N // _NW
_C = 4096
_CHUNKS = _PER_W // _C
_LANES = 16
_BK = 32

_BINS = np.linspace(-3.15, 3.15, 64).astype(np.float32)
_FMAX = np.finfo(np.float32).max
_EBHI = np.concatenate([_BINS, [_FMAX] * 64]).astype(np.float32)
_EBLO = np.concatenate([[-_FMAX], _BINS, [_FMAX] * 63]).astype(np.float32)
_LO = np.float32(_BINS[0])
_INV = np.float32(1.0 / ((3.15 - (-3.15)) / 63))


def _sc_body(x_hbm, eblo_hbm, ebhi_hbm, idx_hbm, x_v, idx_v, eblo_v, ebhi_v):
    wid = lax.axis_index("s") * 2 + lax.axis_index("c")
    base = wid * _PER_W

    pltpu.sync_copy(eblo_hbm, eblo_v)
    pltpu.sync_copy(ebhi_hbm, ebhi_v)

    def chunk_body(ci, carry):
        off = base + ci * _C
        pltpu.sync_copy(x_hbm.at[pl.ds(off, _C)], x_v)

        def vec_body(i, c2):
            xv = x_v[pl.ds(i * _LANES, _LANES)]
            p = (xv - _LO) * _INV
            p = jnp.clip(p, -100.0, 100.0)
            te = jnp.clip(p.astype(jnp.int32) + 1, 0, 64)
            bhi = plsc.load_gather(ebhi_v, [te])
            blo = plsc.load_gather(eblo_v, [te])
            t = te + (xv >= bhi).astype(jnp.int32) - (xv < blo).astype(jnp.int32)
            idx = jnp.clip(t, 1, 64)
            idx = jnp.where(xv != xv, 0, idx)
            idx_v[pl.ds(i * _LANES, _LANES)] = idx
            return c2

        lax.fori_loop(0, _C // _LANES, vec_body, 0)
        pltpu.sync_copy(idx_v, idx_hbm.at[pl.ds(off, _C)])
        return carry

    lax.fori_loop(0, _CHUNKS, chunk_body, 0)


_KPAD = 72


def _tc_body(idx_ref, table_ref, out_ref):
    idxb = idx_ref[...]
    tab = table_ref[...]
    kio = lax.broadcasted_iota(jnp.int32, (_KPAD, 128), 0)
    for s in range(_BK):
        row = lax.slice(idxb, (s, 0), (s + 1, 128))
        oh = (jnp.broadcast_to(row, (_KPAD, 128)) == kio).astype(jnp.float32)
        res = lax.dot_general(
            oh, tab,
            (((0,), (0,)), ((), ())),
            preferred_element_type=jnp.float32,
            precision=lax.Precision.HIGHEST,
        )
        out_ref[s * 128:(s + 1) * 128, :] = res


@jax.jit
def kernel(x, table):
    mesh = plsc.VectorSubcoreMesh(core_axis_name="c", subcore_axis_name="s")
    sc_call = pl.kernel(
        _sc_body,
        out_type=jax.ShapeDtypeStruct((_N,), jnp.int32),
        mesh=mesh,
        compiler_params=pltpu.CompilerParams(
            needs_layout_passes=False, use_tc_tiling_on_sc=False),
        scratch_types=[
            pltpu.VMEM((_C,), jnp.float32),
            pltpu.VMEM((_C,), jnp.int32),
            pltpu.VMEM((128,), jnp.float32),
            pltpu.VMEM((128,), jnp.float32),
        ],
    )
    idx = sc_call(x.reshape(_N), jnp.asarray(_EBLO), jnp.asarray(_EBHI))

    table_pad = jnp.zeros((_KPAD, _D), jnp.float32).at[:65].set(table)
    tc_call = pl.pallas_call(
        _tc_body,
        grid=(_N // (_BK * 128),),
        in_specs=[
            pl.BlockSpec((_BK, 128), lambda i: (i, 0)),
            pl.BlockSpec((_KPAD, _D), lambda i: (0, 0)),
        ],
        out_specs=pl.BlockSpec((_BK * 128, _D), lambda i: (i, 0)),
        out_shape=jax.ShapeDtypeStruct((_N, _D), jnp.float32),
    )
    out = tc_call(idx.reshape(_N // 128, 128), table_pad)
    return out.reshape(_B, _L, _D)


# speedup vs baseline: 3.3017x; 1.0047x over previous
import functools
import numpy as np
import jax
import jax.numpy as jnp
from jax import lax
from jax.experimental import pallas as pl
from jax.experimental.pallas import tpu as pltpu
from jax.experimental.pallas import tpu_sc as plsc

_B, _L, _D = 16384, 200, 32
_N = _B * _L
_NW = 32
_PER_W = _N // _NW
_C = 4096
_CHUNKS = _PER_W // _C
_LANES = 16
_BK = 32

_BINS = np.linspace(-3.15, 3.15, 64).astype(np.float32)
_FMAX = np.finfo(np.float32).max
_EBHI = np.concatenate([_BINS, [_FMAX] * 64]).astype(np.float32)
_EBLO = np.concatenate([[-_FMAX], _BINS, [_FMAX] * 63]).astype(np.float32)
_LO = np.float32(_BINS[0])
_INV = np.float32(1.0 / ((3.15 - (-3.15)) / 63))


def _sc_body(x_hbm, eblo_hbm, ebhi_hbm, idx_hbm, x_v, idx_v, eblo_v, ebhi_v):
    wid = lax.axis_index("s") * 2 + lax.axis_index("c")
    base = wid * _PER_W

    pltpu.sync_copy(eblo_hbm, eblo_v)
    pltpu.sync_copy(ebhi_hbm, ebhi_v)

    def chunk_body(ci, carry):
        off = base + ci * _C
        pltpu.sync_copy(x_hbm.at[pl.ds(off, _C)], x_v)

        def vec_body(i, c2):
            xv = x_v[pl.ds(i * _LANES, _LANES)]
            p = (xv - _LO) * _INV
            p = jnp.clip(p, -100.0, 100.0)
            te = jnp.clip(p.astype(jnp.int32) + 1, 0, 64)
            bhi = plsc.load_gather(ebhi_v, [te])
            blo = plsc.load_gather(eblo_v, [te])
            t = te + (xv >= bhi).astype(jnp.int32) - (xv < blo).astype(jnp.int32)
            idx = jnp.clip(t, 1, 64)
            idx = jnp.where(xv != xv, 0, idx)
            idx_v[pl.ds(i * _LANES, _LANES)] = idx
            return c2

        lax.fori_loop(0, _C // _LANES, vec_body, 0)
        pltpu.sync_copy(idx_v, idx_hbm.at[pl.ds(off, _C)])
        return carry

    lax.fori_loop(0, _CHUNKS, chunk_body, 0)


_KPAD = 72
_TCB = _BK * 128


def _tc_body(idx_ref, table_ref, out_ref):
    row = idx_ref[...].reshape(1, _TCB)
    tab = table_ref[...]
    kio = lax.broadcasted_iota(jnp.int32, (_KPAD, _TCB), 0)
    oh = (jnp.broadcast_to(row, (_KPAD, _TCB)) == kio).astype(jnp.float32)
    out_ref[...] = lax.dot_general(
        oh, tab,
        (((0,), (0,)), ((), ())),
        preferred_element_type=jnp.float32,
        precision=lax.Precision.HIGHEST,
    )


@jax.jit
def kernel(x, table):
    mesh = plsc.VectorSubcoreMesh(core_axis_name="c", subcore_axis_name="s")
    sc_call = pl.kernel(
        _sc_body,
        out_type=jax.ShapeDtypeStruct((_N,), jnp.int32),
        mesh=mesh,
        compiler_params=pltpu.CompilerParams(
            needs_layout_passes=False, use_tc_tiling_on_sc=False),
        scratch_types=[
            pltpu.VMEM((_C,), jnp.float32),
            pltpu.VMEM((_C,), jnp.int32),
            pltpu.VMEM((128,), jnp.float32),
            pltpu.VMEM((128,), jnp.float32),
        ],
    )
    idx = sc_call(x.reshape(_N), jnp.asarray(_EBLO), jnp.asarray(_EBHI))

    table_pad = jnp.zeros((_KPAD, _D), jnp.float32).at[:65].set(table)
    tc_call = pl.pallas_call(
        _tc_body,
        grid=(_N // _TCB,),
        in_specs=[
            pl.BlockSpec((1, 1, _TCB), lambda i: (i, 0, 0)),
            pl.BlockSpec((_KPAD, _D), lambda i: (0, 0)),
        ],
        out_specs=pl.BlockSpec((_TCB, _D), lambda i: (i, 0)),
        out_shape=jax.ShapeDtypeStruct((_N, _D), jnp.float32),
    )
    out = tc_call(idx.reshape(_N // _TCB, 1, _TCB), table_pad)
    return out.reshape(_B, _L, _D)
